# baseline (speedup 1.0000x reference)
import functools

import jax
import jax.numpy as jnp
from jax import lax
from jax.experimental import pallas as pl
from jax.experimental.pallas import tpu as pltpu
from jax.experimental.pallas import tpu_sc as plsc

_D = 64
_NW = 32
_BB = 128
_LANES = 16


def _make_emb_kernel(batch: int, hist: int):
    n_bblk = batch // _BB
    n_hblk = hist // 8
    assert n_bblk == _NW and hist % 8 == 0
    mesh = plsc.VectorSubcoreMesh(core_axis_name="c", subcore_axis_name="s")

    @functools.partial(
        pl.kernel,
        mesh=mesh,
        out_type=jax.ShapeDtypeStruct((hist * 8, n_bblk, 8 * _BB), jnp.float32),
        scratch_types=[
            pltpu.VMEM((n_hblk, 8, _BB), jnp.int32),
            pltpu.VMEM((_BB, _D), jnp.float32),
            pltpu.VMEM((_BB, _D), jnp.float32),
            pltpu.VMEM((8, 8 * _BB), jnp.float32),
            pltpu.VMEM((8, 8 * _BB), jnp.float32),
            pltpu.SemaphoreType.DMA,
            pltpu.SemaphoreType.DMA,
            pltpu.SemaphoreType.DMA,
            pltpu.SemaphoreType.DMA,
        ],
        compiler_params=pltpu.CompilerParams(
            use_tc_tiling_on_sc=False,
            needs_layout_passes=False,
        ),
    )
    def emb(weight_hbm, x3_hbm, out_hbm, idx_v, g0, g1, t0, t1, gs0, gs1, os0, os1):
        bb = lax.axis_index("s") * 2 + lax.axis_index("c")

        for hb in range(n_hblk):
            pltpu.sync_copy(x3_hbm.at[hb * n_bblk + bb], idx_v.at[hb])

        lane = jnp.arange(_LANES, dtype=jnp.int32)
        b_idx = [lane + (b16 * _LANES) for b16 in range(_BB // _LANES)]

        def issue_gather(h, g, gs):
            hb = h // 8
            hi = h - hb * 8
            pltpu.async_copy(weight_hbm.at[idx_v.at[hb, hi]], g, gs)

        def wait_gather(g, gs):
            pltpu.make_async_copy(weight_hbm.at[pl.ds(0, _BB)], g, gs).wait()

        def transpose(g, t):
            for d in range(_D):
                db = jnp.full((_LANES,), d >> 3, jnp.int32)
                dc = jnp.full((_LANES,), d, jnp.int32)
                for b16 in range(_BB // _LANES):
                    v = plsc.load_gather(g, [b_idx[b16], dc])
                    plsc.store_scatter(
                        t, [db, b_idx[b16] + ((d & 7) * _BB)], v
                    )

        def issue_write(h, t, os):
            for db in range(8):
                pltpu.async_copy(t.at[db], out_hbm.at[h * 8 + db, bb], os)

        def drain_write(t, os):
            for db in range(8):
                pltpu.make_async_copy(t.at[db], out_hbm.at[db, bb], os).wait()

        issue_gather(0, g0, gs0)
        issue_gather(1, g1, gs1)
        issue_write(0, t0, os0)
        issue_write(1, t1, os1)

        def step(h, g, t, gs, os):
            wait_gather(g, gs)
            drain_write(t, os)
            transpose(g, t)
            issue_write(h, t, os)
            issue_gather(jnp.minimum(h + 2, hist - 1), g, gs)

        def body(p, carry):
            step(2 * p, g0, t0, gs0, os0)
            step(2 * p + 1, g1, t1, gs1, os1)
            return carry

        lax.fori_loop(0, hist // 2, body, 0)
        wait_gather(g0, gs0)
        wait_gather(g1, gs1)
        drain_write(t0, os0)
        drain_write(t1, os1)

    return emb


@jax.jit
def kernel(x, weight):
    batch, hist = x.shape
    vocab, d = weight.shape
    x3 = (
        x.T.astype(jnp.int32)
        .reshape(hist // 8, 8, batch // _BB, _BB)
        .transpose(0, 2, 1, 3)
        .reshape((hist // 8) * (batch // _BB), 8, _BB)
    )
    out3 = _make_emb_kernel(batch, hist)(weight, x3)
    out = (
        out3.reshape(hist, 8, batch // _BB, 8, _BB)
        .transpose(0, 1, 3, 2, 4)
        .reshape(hist, d, batch)
        .transpose(2, 0, 1)
    )
    return out


# speedup vs baseline: 1.9609x; 1.9609x over previous
import functools

import jax
import jax.numpy as jnp
from jax import lax
from jax.experimental import pallas as pl
from jax.experimental.pallas import tpu as pltpu
from jax.experimental.pallas import tpu_sc as plsc

_D = 64
_NW = 32
_BB = 128
_LANES = 16


def _make_emb_kernel(batch: int, hist: int):
    n_bblk = batch // _BB
    n_hblk = hist // 8
    assert n_bblk == _NW and hist % 8 == 0
    mesh = plsc.VectorSubcoreMesh(core_axis_name="c", subcore_axis_name="s")

    @functools.partial(
        pl.kernel,
        mesh=mesh,
        out_type=jax.ShapeDtypeStruct((hist * 8, n_bblk, 8, _BB), jnp.float32),
        scratch_types=[
            pltpu.VMEM((n_hblk, 8, _BB), jnp.int32),
            pltpu.VMEM((_BB, _D), jnp.float32),
            pltpu.VMEM((_BB, _D), jnp.float32),
            pltpu.VMEM((_D, _BB), jnp.float32),
            pltpu.VMEM((_D, _BB), jnp.float32),
            pltpu.SemaphoreType.DMA,
            pltpu.SemaphoreType.DMA,
            pltpu.SemaphoreType.DMA,
            pltpu.SemaphoreType.DMA,
        ],
        compiler_params=pltpu.CompilerParams(
            use_tc_tiling_on_sc=False,
            needs_layout_passes=False,
        ),
    )
    def emb(weight_hbm, x3_hbm, out_hbm, idx_v, g0, g1, t0, t1, gs0, gs1, os0, os1):
        bb = lax.axis_index("s") * 2 + lax.axis_index("c")

        for hb in range(n_hblk):
            pltpu.sync_copy(x3_hbm.at[hb * n_bblk + bb], idx_v.at[hb])

        lane = jnp.arange(_LANES, dtype=jnp.int32)
        b_idx = [lane + (b16 * _LANES) for b16 in range(_BB // _LANES)]

        def issue_gather(h, g, gs):
            hb = h // 8
            hi = h - hb * 8
            pltpu.async_copy(weight_hbm.at[idx_v.at[hb, hi]], g, gs)

        def wait_gather(g, gs):
            pltpu.make_async_copy(weight_hbm.at[pl.ds(0, _BB)], g, gs).wait()

        def transpose(g, t):
            def diag(q, carry):
                d0 = (q >> 4) << 4
                s = q & (_LANES - 1)
                dv = ((lane + s) & (_LANES - 1)) + d0
                for b16 in range(_BB // _LANES):
                    v = plsc.load_gather(g, [b_idx[b16], dv])
                    plsc.store_scatter(t, [dv, b_idx[b16]], v)
                return carry

            lax.fori_loop(0, _D, diag, 0)

        def issue_write(h, t, os):
            for db in range(8):
                pltpu.async_copy(
                    t.at[pl.ds(db * 8, 8)], out_hbm.at[h * 8 + db, bb], os
                )

        def drain_write(t, os):
            for db in range(8):
                pltpu.make_async_copy(
                    t.at[pl.ds(db * 8, 8)], out_hbm.at[db, bb], os
                ).wait()

        issue_gather(0, g0, gs0)
        issue_gather(1, g1, gs1)
        issue_write(0, t0, os0)
        issue_write(1, t1, os1)

        def step(h, g, t, gs, os):
            wait_gather(g, gs)
            drain_write(t, os)
            transpose(g, t)
            issue_write(h, t, os)
            issue_gather(jnp.minimum(h + 2, hist - 1), g, gs)

        def body(p, carry):
            step(2 * p, g0, t0, gs0, os0)
            step(2 * p + 1, g1, t1, gs1, os1)
            return carry

        lax.fori_loop(0, hist // 2, body, 0)
        wait_gather(g0, gs0)
        wait_gather(g1, gs1)
        drain_write(t0, os0)
        drain_write(t1, os1)

    return emb


@jax.jit
def kernel(x, weight):
    batch, hist = x.shape
    vocab, d = weight.shape
    x3 = (
        x.T.astype(jnp.int32)
        .reshape(hist // 8, 8, batch // _BB, _BB)
        .transpose(0, 2, 1, 3)
        .reshape((hist // 8) * (batch // _BB), 8, _BB)
    )
    out4 = _make_emb_kernel(batch, hist)(weight, x3)
    out = (
        out4.reshape(hist, 8, batch // _BB, 8, _BB)
        .transpose(0, 1, 3, 2, 4)
        .reshape(hist, d, batch)
        .transpose(2, 0, 1)
    )
    return out


# speedup vs baseline: 2.5318x; 1.2911x over previous
import functools

import jax
import jax.numpy as jnp
from jax import lax
from jax.experimental import pallas as pl
from jax.experimental.pallas import tpu as pltpu
from jax.experimental.pallas import tpu_sc as plsc

_D = 64
_NW = 32
_BB = 128
_LANES = 16


def _make_emb_kernel(batch: int, hist: int):
    n_bblk = batch // _BB
    n_hblk = hist // 8
    assert n_bblk == _NW and hist % 8 == 0
    mesh = plsc.VectorSubcoreMesh(core_axis_name="c", subcore_axis_name="s")

    @functools.partial(
        pl.kernel,
        mesh=mesh,
        out_type=jax.ShapeDtypeStruct((hist * 8, n_bblk, 8, _BB), jnp.float32),
        scratch_types=[
            pltpu.VMEM((n_hblk, 8, _BB), jnp.int32),
            pltpu.VMEM((_BB, _D), jnp.float32),
            pltpu.VMEM((_BB, _D), jnp.float32),
            pltpu.VMEM((_D, _BB), jnp.float32),
            pltpu.VMEM((_D, _BB), jnp.float32),
            pltpu.SemaphoreType.DMA,
            pltpu.SemaphoreType.DMA,
            pltpu.SemaphoreType.DMA,
            pltpu.SemaphoreType.DMA,
        ],
        compiler_params=pltpu.CompilerParams(
            use_tc_tiling_on_sc=False,
            needs_layout_passes=False,
        ),
    )
    def emb(weight_hbm, x3_hbm, out_hbm, idx_v, g0, g1, t0, t1, gs0, gs1, os0, os1):
        bb = lax.axis_index("s") * 2 + lax.axis_index("c")

        for hb in range(n_hblk):
            pltpu.sync_copy(x3_hbm.at[hb * n_bblk + bb], idx_v.at[hb])

        lane = jnp.arange(_LANES, dtype=jnp.int32)
        b_idx = [lane + (b16 * _LANES) for b16 in range(_BB // _LANES)]

        def issue_gather(h, g, gs):
            hb = h // 8
            hi = h - hb * 8
            pltpu.async_copy(weight_hbm.at[idx_v.at[hb, hi]], g, gs)

        def wait_gather(g, gs):
            pltpu.make_async_copy(weight_hbm.at[pl.ds(0, _BB)], g, gs).wait()

        def transpose(g, t):
            def diag(q, carry):
                q0 = q * 2
                d0 = (q0 >> 4) << 4
                s = q0 & (_LANES - 1)
                dvs = [((lane + s + u) & (_LANES - 1)) + d0 for u in range(2)]
                vs = [
                    plsc.load_gather(g, [b_idx[b16], dv])
                    for dv in dvs
                    for b16 in range(_BB // _LANES)
                ]
                i = 0
                for dv in dvs:
                    for b16 in range(_BB // _LANES):
                        plsc.store_scatter(t, [dv, b_idx[b16]], vs[i])
                        i += 1
                return carry

            lax.fori_loop(0, _D // 2, diag, 0)

        def issue_write(h, t, os):
            for db in range(8):
                pltpu.async_copy(
                    t.at[pl.ds(db * 8, 8)], out_hbm.at[h * 8 + db, bb], os
                )

        def drain_write(t, os):
            for db in range(8):
                pltpu.make_async_copy(
                    t.at[pl.ds(db * 8, 8)], out_hbm.at[db, bb], os
                ).wait()

        issue_gather(0, g0, gs0)
        issue_gather(1, g1, gs1)
        issue_write(0, t0, os0)
        issue_write(1, t1, os1)

        def step(h, g, t, gs, os):
            wait_gather(g, gs)
            drain_write(t, os)
            transpose(g, t)
            issue_write(h, t, os)
            issue_gather(jnp.minimum(h + 2, hist - 1), g, gs)

        def body(p, carry):
            step(2 * p, g0, t0, gs0, os0)
            step(2 * p + 1, g1, t1, gs1, os1)
            return carry

        lax.fori_loop(0, hist // 2, body, 0)
        wait_gather(g0, gs0)
        wait_gather(g1, gs1)
        drain_write(t0, os0)
        drain_write(t1, os1)

    return emb


@jax.jit
def kernel(x, weight):
    batch, hist = x.shape
    vocab, d = weight.shape
    x3 = (
        x.T.astype(jnp.int32)
        .reshape(hist // 8, 8, batch // _BB, _BB)
        .transpose(0, 2, 1, 3)
        .reshape((hist // 8) * (batch // _BB), 8, _BB)
    )
    out4 = _make_emb_kernel(batch, hist)(weight, x3)
    out = (
        out4.reshape(hist, 8, batch // _BB, 8, _BB)
        .transpose(0, 1, 3, 2, 4)
        .reshape(hist, d, batch)
        .transpose(2, 0, 1)
    )
    return out
